# Initial kernel scaffold; baseline (speedup 1.0000x reference)
import jax
import jax.numpy as jnp
from jax.experimental import pallas as pl


def kernel(input_ids, segment_ids, word_emb, pos_emb, type_emb, ln_gamma, ln_beta, W_proj, b_proj):
    raise NotImplementedError("write your pallas kernel here")


# speedup vs baseline: 1.8196x; 1.8196x over previous
import functools

import jax
import jax.numpy as jnp
from jax import lax
from jax.experimental import pallas as pl
from jax.experimental.pallas import tpu as pltpu
from jax.experimental.pallas import tpu_sc as plsc


def _sc_gather(table, flat_ids):
    bs = flat_ids.shape[0]
    d = table.shape[1]
    info = plsc.get_sparse_core_info()
    nw = info.num_cores * info.num_subcores
    per_w = bs // nw
    chunk = 64
    n_chunks = per_w // chunk
    mesh = plsc.VectorSubcoreMesh(core_axis_name="c", subcore_axis_name="s")

    @functools.partial(
        pl.kernel,
        mesh=mesh,
        out_type=jax.ShapeDtypeStruct((bs, d), jnp.float32),
        scratch_types=[
            pltpu.VMEM((per_w,), jnp.int32),
            pltpu.VMEM((chunk, d), jnp.float32),
            pltpu.VMEM((chunk, d), jnp.float32),
            pltpu.SemaphoreType.DMA,
            pltpu.SemaphoreType.DMA,
        ],
    )
    def k(ids_hbm, table_hbm, out_hbm, idx_v, rows0, rows1, sem0, sem1):
        wid = lax.axis_index("s") * info.num_cores + lax.axis_index("c")
        base = wid * per_w
        pltpu.sync_copy(ids_hbm.at[pl.ds(base, per_w)], idx_v)
        rows = (rows0, rows1)
        sems = (sem0, sem1)
        pltpu.async_copy(table_hbm.at[idx_v.at[pl.ds(0, chunk)]], rows[0], sems[0])
        for c in range(n_chunks):
            if c + 1 < n_chunks:
                pltpu.async_copy(
                    table_hbm.at[idx_v.at[pl.ds((c + 1) * chunk, chunk)]],
                    rows[(c + 1) % 2], sems[(c + 1) % 2])
            pltpu.make_async_copy(
                table_hbm.at[idx_v.at[pl.ds(c * chunk, chunk)]],
                rows[c % 2], sems[c % 2]).wait()
            pltpu.sync_copy(rows[c % 2], out_hbm.at[pl.ds(base + c * chunk, chunk)])

    return k(flat_ids, table)


def _tc_body(g_ref, pos_ref, seg_ref, type_ref, gamma_ref, beta_ref,
             w_ref, b_ref, out_ref):
    t0 = type_ref[0, :][None, :]
    t1 = type_ref[1, :][None, :]
    sf = seg_ref[0, 0, :][:, None]
    emb = g_ref[...] + pos_ref[...] + (t0 + sf * (t1 - t0))
    mu = jnp.mean(emb, axis=1, keepdims=True)
    xc = emb - mu
    var = jnp.mean(xc * xc, axis=1, keepdims=True)
    y = xc * lax.rsqrt(var + 1e-12) * gamma_ref[0, :] + beta_ref[0, :]
    out_ref[...] = jnp.dot(y, w_ref[...],
                           preferred_element_type=jnp.float32) + b_ref[0, :]


def _tc_fuse(gathered, pos_emb, segf, type_emb, gamma, beta, w, b):
    bs, d = gathered.shape
    nseq, _, s = segf.shape
    grid = (nseq,)
    return pl.pallas_call(
        _tc_body,
        grid=grid,
        in_specs=[
            pl.BlockSpec((s, d), lambda i: (i, 0)),
            pl.BlockSpec((s, d), lambda i: (0, 0)),
            pl.BlockSpec((1, 1, s), lambda i: (i, 0, 0)),
            pl.BlockSpec((2, d), lambda i: (0, 0)),
            pl.BlockSpec((1, d), lambda i: (0, 0)),
            pl.BlockSpec((1, d), lambda i: (0, 0)),
            pl.BlockSpec((d, d), lambda i: (0, 0)),
            pl.BlockSpec((1, d), lambda i: (0, 0)),
        ],
        out_specs=pl.BlockSpec((s, d), lambda i: (i, 0)),
        out_shape=jax.ShapeDtypeStruct((bs, d), jnp.float32),
    )(gathered, pos_emb, segf, type_emb, gamma, beta, w, b)


def kernel(input_ids, segment_ids, word_emb, pos_emb, type_emb,
           ln_gamma, ln_beta, W_proj, b_proj):
    nb, s = input_ids.shape
    d = word_emb.shape[1]
    flat_ids = input_ids.reshape(-1)
    gathered = _sc_gather(word_emb, flat_ids)
    segf = segment_ids.astype(jnp.float32).reshape(nb, 1, s)
    out = _tc_fuse(gathered, pos_emb, segf, type_emb,
                   ln_gamma.reshape(1, d), ln_beta.reshape(1, d),
                   W_proj, b_proj.reshape(1, d))
    return out.reshape(nb, s, d)
